# baseline (speedup 1.0000x reference)
import functools

import jax
import jax.numpy as jnp
from jax import lax
from jax.experimental import pallas as pl
from jax.experimental.pallas import tpu as pltpu
from jax.experimental.pallas import tpu_sc as plsc

NE = 100000
NR = 1000
D = 64
NT = 1600000

NW = 32
CHUNK = NT // NW
HWIN = 2000
NHW = CHUNK // HWIN

EMPTY = 0x7FFFFFFF
SIGN = -2147483648

MA_BIG = 10
MA_BIG_N, MA_SMALL_N = 6256, 6240
MA_BIG_TOTAL = MA_BIG * MA_BIG_N

N_BIG = 10
BIG_N, BIG_W, BIG_WINS = 3136, 224, 14
SMALL_N, SMALL_W, SMALL_WINS = 3120, 240, 13
BIG_TOTAL = N_BIG * BIG_N
SPP = 248

_MESH = plsc.VectorSubcoreMesh(core_axis_name="c", subcore_axis_name="s")
_PARAMS = pltpu.CompilerParams(needs_layout_passes=False,
                               use_tc_tiling_on_sc=False)


@functools.partial(
    pl.kernel,
    out_type=jax.ShapeDtypeStruct((NW * NE,), jnp.int32),
    mesh=_MESH,
    compiler_params=_PARAMS,
    scratch_types=[
        pltpu.VMEM((NE,), jnp.int32),
        pltpu.VMEM((3 * HWIN,), jnp.int32),
        pltpu.VMEM((3 * HWIN,), jnp.int32),
        pltpu.SemaphoreType.DMA,
        pltpu.SemaphoreType.DMA,
    ],
)
def _phase_a(h_hbm, r_hbm, t_hbm, wall_hbm, table, bufA, bufB, semA, semB):
    cid = lax.axis_index("c")
    sid = lax.axis_index("s")
    wid = sid * 2 + cid
    lanes = lax.iota(jnp.int32, 16)

    def init(i, carry):
        table[pl.ds(i * 16, 16)] = jnp.full((16,), EMPTY, jnp.int32)
        return carry

    lax.fori_loop(0, NE // 16, init, 0)

    base = wid * CHUNK
    widtag = (31 - wid) << 27

    def start_win(w, buf, sem):
        s = base + w * HWIN
        for c, col in enumerate((h_hbm, r_hbm, t_hbm)):
            pltpu.async_copy(col.at[pl.ds(s, HWIN)],
                             buf.at[pl.ds(c * HWIN, HWIN)], sem)

    def wait_win(buf, sem):
        for c in range(3):
            pltpu.make_async_copy(h_hbm.at[pl.ds(base, HWIN)],
                                  buf.at[pl.ds(c * HWIN, HWIN)], sem).wait()

    def scan_buf(w, buf):
        def vstep(i, c2):
            sl = pl.ds(i * 16, 16)
            h = buf[sl]
            rl = buf[pl.ds(HWIN + i * 16, 16)]
            tl = buf[pl.ds(2 * HWIN + i * 16, 16)]
            key = (h << 4) | lanes
            pay = (rl << 17) | tl
            ks, vs = plsc.sort_key_val(key, pay)
            hs = ks >> 4
            nxt = hs.at[jnp.minimum(lanes + 1, 15)].get(
                mode="promise_in_bounds")
            mask = (hs != nxt) | (lanes == 15)
            val = (widtag | vs) ^ SIGN
            plsc.store_scatter(table, [hs], val, mask=mask)
            return c2

        lax.fori_loop(0, HWIN // 16, vstep, 0, unroll=8)

    bufs = (bufA, bufB)
    sems = (semA, semB)
    start_win(0, bufA, semA)
    for w in range(NHW):
        p = w % 2
        wait_win(bufs[p], sems[p])
        if w + 1 < NHW:
            start_win(w + 1, bufs[1 - p], sems[1 - p])
        scan_buf(w, bufs[p])
    pltpu.sync_copy(table, wall_hbm.at[pl.ds(wid * NE, NE)])


@functools.partial(
    pl.kernel,
    out_type=jax.ShapeDtypeStruct((NE, D), jnp.float32),
    mesh=_MESH,
    compiler_params=_PARAMS,
    scratch_types=[
        pltpu.VMEM((BIG_N,), jnp.int32),
        pltpu.VMEM((BIG_N,), jnp.int32),
        pltpu.VMEM((BIG_N,), jnp.int32),
        pltpu.VMEM((BIG_W,), jnp.int32),
        pltpu.VMEM((BIG_W,), jnp.int32),
        pltpu.VMEM((SMALL_W,), jnp.int32),
        pltpu.VMEM((SMALL_W,), jnp.int32),
        pltpu.VMEM((BIG_W,), jnp.int32),
        pltpu.VMEM((BIG_W,), jnp.int32),
        pltpu.VMEM((SMALL_W,), jnp.int32),
        pltpu.VMEM((SMALL_W,), jnp.int32),
        pltpu.VMEM((BIG_W,), jnp.int32),
        pltpu.VMEM((BIG_W,), jnp.int32),
        pltpu.VMEM((SMALL_W,), jnp.int32),
        pltpu.VMEM((SMALL_W,), jnp.int32),
        pltpu.VMEM((SMALL_W, D), jnp.float32),
        pltpu.VMEM((SMALL_W, D), jnp.float32),
        pltpu.VMEM((SMALL_W, D), jnp.float32),
        pltpu.VMEM((SMALL_W, D), jnp.float32),
        pltpu.VMEM_SHARED((16 * 2 * SPP, D), jnp.float32),
        pltpu.SemaphoreType.DMA,
        pltpu.SemaphoreType.DMA,
        pltpu.SemaphoreType.DMA,
        pltpu.SemaphoreType.DMA,
        pltpu.SemaphoreType.DMA,
        pltpu.SemaphoreType.DMA,
        pltpu.SemaphoreType.DMA,
        pltpu.SemaphoreType.DMA,
    ],
)
def _phase_b(mem_hbm, rel_hbm, wall_hbm, out_hbm,
             wfull, stg0, stg1, tgtA0, tgtA1, tgtB0, tgtB1,
             rwA0, rwA1, rwB0, rwB1, twA0, twA1, twB0, twB1,
             rel0, rel1, tb0, tb1, spmem,
             semr0, semr1, semt0, semt1, semm0, semm1, semo0, semo1):
    cid = lax.axis_index("c")
    sid = lax.axis_index("s")
    wid = sid * 2 + cid
    lanes = lax.iota(jnp.int32, 16)
    rels = (rel0, rel1)
    tbs = (tb0, tb1)
    semrs = (semr0, semr1)
    semts = (semt0, semt1)
    semms = (semm0, semm1)
    semos = (semo0, semo1)

    def run(nwin, wsize, e0_fn, tgts, rws, tws):
        nv = wsize // 16
        n_range = nwin * wsize
        e0r = e0_fn(0)
        stgs = (stg0, stg1)

        cw = pltpu.async_copy(wall_hbm.at[pl.ds(e0r, n_range)],
                              wfull.at[pl.ds(0, n_range)], semt0)
        ck = pltpu.async_copy(wall_hbm.at[pl.ds(NE + e0r, n_range)],
                              stg0.at[pl.ds(0, n_range)], semr0)
        cw.wait()
        for k in range(1, NW):
            p = (k - 1) % 2
            if k + 1 < NW:
                pltpu.async_copy(
                    wall_hbm.at[pl.ds((k + 1) * NE + e0r, n_range)],
                    stgs[1 - p].at[pl.ds(0, n_range)],
                    semrs[1 - p])
            if k == 1:
                ck.wait()
            else:
                pltpu.make_async_copy(
                    wall_hbm.at[pl.ds(k * NE + e0r, n_range)],
                    stgs[p].at[pl.ds(0, n_range)], semrs[p]).wait()
            stg = stgs[p]

            def mstep(j, carry):
                col = pl.ds(j * 16, 16)
                wfull[col] = jnp.minimum(wfull[col], stg[col])
                return carry

            lax.fori_loop(0, n_range // 16, mstep, 0, unroll=4)

        out_h = {}

        def stage1(w):
            p = w % 2
            e0 = e0_fn(w)
            spb = sid * (2 * SPP) + p * SPP
            rw, tw, tgt = rws[p], tws[p], tgts[p]
            if w - 2 in out_h:
                out_h[w - 2].wait()

            def jstep(j, carry):
                col = pl.ds(j * 16, 16)
                acc = wfull[pl.ds(w * wsize + j * 16, 16)]
                nowin = acc == EMPTY
                u = acc ^ SIGN
                r = lax.shift_right_logical(u, 17) & 0x3FF
                t = u & 0x1FFFF
                rw[col] = jnp.where(nowin, 0, r)
                tw[col] = jnp.where(nowin, 0, t)
                eloc = j * 16 + lanes
                tgt[col] = spb + jnp.where(nowin, 240 + (lanes & 7), eloc)
                return carry

            lax.fori_loop(0, nv, jstep, 0, unroll=2)

            cr = pltpu.async_copy(rel_hbm.at[rw],
                                  rels[p].at[pl.ds(0, wsize)], semrs[p])
            ct = pltpu.async_copy(mem_hbm.at[tw],
                                  tbs[p].at[pl.ds(0, wsize)], semts[p])
            cm = pltpu.async_copy(mem_hbm.at[pl.ds(e0, wsize)],
                                  spmem.at[pl.ds(spb, wsize)], semms[p])
            return (cr, ct, cm)

        def stage2(w, handles):
            p = w % 2
            e0 = e0_fn(w)
            spb = sid * (2 * SPP) + p * SPP
            rel, tb, tgt = rels[p], tbs[p], tgts[p]
            for c in handles:
                c.wait()

            def dstep(j, carry):
                for c in range(D // 16):
                    col = pl.ds(c * 16, 16)
                    rel[j, col] = 0.1 * (rel[j, col] - tb[j, col])
                return carry

            lax.fori_loop(0, wsize, dstep, 0, unroll=2)

            pltpu.sync_copy(rel.at[pl.ds(0, wsize)], spmem.at[tgt],
                            add=True)
            out_h[w] = pltpu.async_copy(spmem.at[pl.ds(spb, wsize)],
                                        out_hbm.at[pl.ds(e0, wsize)],
                                        semos[p])

        handles = stage1(0)
        for w in range(1, nwin):
            nh = stage1(w)
            stage2(w - 1, handles)
            handles = nh
        stage2(nwin - 1, handles)
        out_h[nwin - 2].wait()
        out_h[nwin - 1].wait()

    @pl.when(wid < N_BIG)
    def _big():
        run(BIG_WINS, BIG_W, lambda w: wid * BIG_N + w * BIG_W,
            (tgtA0, tgtA1), (rwA0, rwA1), (twA0, twA1))

    @pl.when(wid >= N_BIG)
    def _small():
        run(SMALL_WINS, SMALL_W,
            lambda w: BIG_TOTAL + (wid - N_BIG) * SMALL_N + w * SMALL_W,
            (tgtB0, tgtB1), (rwB0, rwB1), (twB0, twB1))


def kernel(memory, rel_table, triples):
    wall = _phase_a(triples[:, 0], triples[:, 1], triples[:, 2])
    return _phase_b(memory, rel_table, wall)


# speedup vs baseline: 1.0115x; 1.0115x over previous
import functools

import jax
import jax.numpy as jnp
from jax import lax
from jax.experimental import pallas as pl
from jax.experimental.pallas import tpu as pltpu
from jax.experimental.pallas import tpu_sc as plsc

NE = 100000
NR = 1000
D = 64
NT = 1600000

NW = 32
CHUNK = NT // NW
HWIN = 2000
NHW = CHUNK // HWIN

EMPTY = 0x7FFFFFFF
SIGN = -2147483648

MA_BIG = 10
MA_BIG_N, MA_SMALL_N = 6256, 6240
MA_BIG_TOTAL = MA_BIG * MA_BIG_N

N_BIG = 10
BIG_N, BIG_W, BIG_WINS = 3136, 224, 14
SMALL_N, SMALL_W, SMALL_WINS = 3120, 240, 13
BIG_TOTAL = N_BIG * BIG_N
SPP = 248

_MESH = plsc.VectorSubcoreMesh(core_axis_name="c", subcore_axis_name="s")
_PARAMS = pltpu.CompilerParams(needs_layout_passes=False,
                               use_tc_tiling_on_sc=False)


@functools.partial(
    pl.kernel,
    out_type=jax.ShapeDtypeStruct((NW * NE,), jnp.int32),
    mesh=_MESH,
    compiler_params=_PARAMS,
    scratch_types=[
        pltpu.VMEM((NE,), jnp.int32),
        pltpu.VMEM((3 * HWIN,), jnp.int32),
        pltpu.VMEM((3 * HWIN,), jnp.int32),
        pltpu.SemaphoreType.DMA,
        pltpu.SemaphoreType.DMA,
    ],
)
def _phase_a(h_hbm, r_hbm, t_hbm, wall_hbm, table, bufA, bufB, semA, semB):
    cid = lax.axis_index("c")
    sid = lax.axis_index("s")
    wid = sid * 2 + cid
    lanes = lax.iota(jnp.int32, 16)

    def init(i, carry):
        table[pl.ds(i * 16, 16)] = jnp.full((16,), EMPTY, jnp.int32)
        return carry

    lax.fori_loop(0, NE // 16, init, 0)

    base = wid * CHUNK
    widtag = (31 - wid) << 27

    def start_win(w, buf, sem):
        s = base + w * HWIN
        for c, col in enumerate((h_hbm, r_hbm, t_hbm)):
            pltpu.async_copy(col.at[pl.ds(s, HWIN)],
                             buf.at[pl.ds(c * HWIN, HWIN)], sem)

    def wait_win(buf, sem):
        for c in range(3):
            pltpu.make_async_copy(h_hbm.at[pl.ds(base, HWIN)],
                                  buf.at[pl.ds(c * HWIN, HWIN)], sem).wait()

    def scan_buf(w, buf):
        def vstep(i, c2):
            sl = pl.ds(i * 16, 16)
            h = buf[sl]
            rl = buf[pl.ds(HWIN + i * 16, 16)]
            tl = buf[pl.ds(2 * HWIN + i * 16, 16)]
            key = (h << 4) | lanes
            pay = (rl << 17) | tl
            ks, vs = plsc.sort_key_val(key, pay)
            hs = ks >> 4
            nxt = hs.at[jnp.minimum(lanes + 1, 15)].get(
                mode="promise_in_bounds")
            mask = (hs != nxt) | (lanes == 15)
            val = (widtag | vs) ^ SIGN
            plsc.store_scatter(table, [hs], val, mask=mask)
            return c2

        lax.fori_loop(0, HWIN // 16, vstep, 0, unroll=8)

    start_win(0, bufA, semA)

    def win2(w, carry):
        @pl.when(w % 2 == 0)
        def _even():
            wait_win(bufA, semA)

            @pl.when(w + 1 < NHW)
            def _():
                start_win(w + 1, bufB, semB)

            scan_buf(w, bufA)

        @pl.when(w % 2 == 1)
        def _odd():
            wait_win(bufB, semB)

            @pl.when(w + 1 < NHW)
            def _():
                start_win(w + 1, bufA, semA)

            scan_buf(w, bufB)

        return carry

    lax.fori_loop(0, NHW, win2, 0)
    pltpu.sync_copy(table, wall_hbm.at[pl.ds(wid * NE, NE)])


@functools.partial(
    pl.kernel,
    out_type=jax.ShapeDtypeStruct((NE, D), jnp.float32),
    mesh=_MESH,
    compiler_params=_PARAMS,
    scratch_types=[
        pltpu.VMEM((BIG_N,), jnp.int32),
        pltpu.VMEM((BIG_N,), jnp.int32),
        pltpu.VMEM((BIG_N,), jnp.int32),
        pltpu.VMEM((BIG_W,), jnp.int32),
        pltpu.VMEM((BIG_W,), jnp.int32),
        pltpu.VMEM((SMALL_W,), jnp.int32),
        pltpu.VMEM((SMALL_W,), jnp.int32),
        pltpu.VMEM((BIG_W,), jnp.int32),
        pltpu.VMEM((BIG_W,), jnp.int32),
        pltpu.VMEM((SMALL_W,), jnp.int32),
        pltpu.VMEM((SMALL_W,), jnp.int32),
        pltpu.VMEM((BIG_W,), jnp.int32),
        pltpu.VMEM((BIG_W,), jnp.int32),
        pltpu.VMEM((SMALL_W,), jnp.int32),
        pltpu.VMEM((SMALL_W,), jnp.int32),
        pltpu.VMEM((SMALL_W, D), jnp.float32),
        pltpu.VMEM((SMALL_W, D), jnp.float32),
        pltpu.VMEM((SMALL_W, D), jnp.float32),
        pltpu.VMEM((SMALL_W, D), jnp.float32),
        pltpu.VMEM_SHARED((16 * 2 * SPP, D), jnp.float32),
        pltpu.VMEM_SHARED((NR, D), jnp.float32),
        pltpu.SemaphoreType.DMA,
        pltpu.SemaphoreType.DMA,
        pltpu.SemaphoreType.DMA,
        pltpu.SemaphoreType.DMA,
        pltpu.SemaphoreType.DMA,
        pltpu.SemaphoreType.DMA,
        pltpu.SemaphoreType.DMA,
        pltpu.SemaphoreType.DMA,
    ],
)
def _phase_b(mem_hbm, rel_hbm, wall_hbm, out_hbm,
             wfull, stg0, stg1, tgtA0, tgtA1, tgtB0, tgtB1,
             rwA0, rwA1, rwB0, rwB1, twA0, twA1, twB0, twB1,
             rel0, rel1, tb0, tb1, spmem, relsp,
             semr0, semr1, semt0, semt1, semm0, semm1, semo0, semo1):
    cid = lax.axis_index("c")
    sid = lax.axis_index("s")
    wid = sid * 2 + cid
    lanes = lax.iota(jnp.int32, 16)

    @pl.when(sid == 0)
    def _stage_rel():
        pltpu.sync_copy(rel_hbm, relsp)

    plsc.subcore_barrier()
    rels = (rel0, rel1)
    tbs = (tb0, tb1)
    semrs = (semr0, semr1)
    semts = (semt0, semt1)
    semms = (semm0, semm1)
    semos = (semo0, semo1)

    def run(nwin, wsize, e0_fn, tgts, rws, tws):
        nv = wsize // 16
        n_range = nwin * wsize
        e0r = e0_fn(0)
        stgs = (stg0, stg1)

        cw = pltpu.async_copy(wall_hbm.at[pl.ds(e0r, n_range)],
                              wfull.at[pl.ds(0, n_range)], semt0)
        ck = pltpu.async_copy(wall_hbm.at[pl.ds(NE + e0r, n_range)],
                              stg0.at[pl.ds(0, n_range)], semr0)
        cw.wait()
        for k in range(1, NW):
            p = (k - 1) % 2
            if k + 1 < NW:
                pltpu.async_copy(
                    wall_hbm.at[pl.ds((k + 1) * NE + e0r, n_range)],
                    stgs[1 - p].at[pl.ds(0, n_range)],
                    semrs[1 - p])
            if k == 1:
                ck.wait()
            else:
                pltpu.make_async_copy(
                    wall_hbm.at[pl.ds(k * NE + e0r, n_range)],
                    stgs[p].at[pl.ds(0, n_range)], semrs[p]).wait()
            stg = stgs[p]

            def mstep(j, carry):
                col = pl.ds(j * 16, 16)
                wfull[col] = jnp.minimum(wfull[col], stg[col])
                return carry

            lax.fori_loop(0, n_range // 16, mstep, 0, unroll=4)

        out_h = {}

        def stage1(w):
            p = w % 2
            e0 = e0_fn(w)
            spb = sid * (2 * SPP) + p * SPP
            rw, tw, tgt = rws[p], tws[p], tgts[p]
            if w - 2 in out_h:
                out_h[w - 2].wait()

            def jstep(j, carry):
                col = pl.ds(j * 16, 16)
                acc = wfull[pl.ds(w * wsize + j * 16, 16)]
                nowin = acc == EMPTY
                u = acc ^ SIGN
                r = lax.shift_right_logical(u, 17) & 0x3FF
                t = u & 0x1FFFF
                rw[col] = jnp.where(nowin, 0, r)
                tw[col] = jnp.where(nowin, 0, t)
                eloc = j * 16 + lanes
                tgt[col] = spb + jnp.where(nowin, 240 + (lanes & 7), eloc)
                return carry

            lax.fori_loop(0, nv, jstep, 0, unroll=2)

            cr = pltpu.async_copy(relsp.at[rw],
                                  rels[p].at[pl.ds(0, wsize)], semrs[p])
            ct = pltpu.async_copy(mem_hbm.at[tw],
                                  tbs[p].at[pl.ds(0, wsize)], semts[p])
            cm = pltpu.async_copy(mem_hbm.at[pl.ds(e0, wsize)],
                                  spmem.at[pl.ds(spb, wsize)], semms[p])
            return (cr, ct, cm)

        def stage2(w, handles):
            p = w % 2
            e0 = e0_fn(w)
            spb = sid * (2 * SPP) + p * SPP
            rel, tb, tgt = rels[p], tbs[p], tgts[p]
            for c in handles:
                c.wait()

            def dstep(j, carry):
                for c in range(D // 16):
                    col = pl.ds(c * 16, 16)
                    rel[j, col] = 0.1 * (rel[j, col] - tb[j, col])
                return carry

            lax.fori_loop(0, wsize, dstep, 0, unroll=2)

            pltpu.sync_copy(rel.at[pl.ds(0, wsize)], spmem.at[tgt],
                            add=True)
            out_h[w] = pltpu.async_copy(spmem.at[pl.ds(spb, wsize)],
                                        out_hbm.at[pl.ds(e0, wsize)],
                                        semos[p])

        handles = stage1(0)
        for w in range(1, nwin):
            nh = stage1(w)
            stage2(w - 1, handles)
            handles = nh
        stage2(nwin - 1, handles)
        out_h[nwin - 2].wait()
        out_h[nwin - 1].wait()

    @pl.when(wid < N_BIG)
    def _big():
        run(BIG_WINS, BIG_W, lambda w: wid * BIG_N + w * BIG_W,
            (tgtA0, tgtA1), (rwA0, rwA1), (twA0, twA1))

    @pl.when(wid >= N_BIG)
    def _small():
        run(SMALL_WINS, SMALL_W,
            lambda w: BIG_TOTAL + (wid - N_BIG) * SMALL_N + w * SMALL_W,
            (tgtB0, tgtB1), (rwB0, rwB1), (twB0, twB1))


def kernel(memory, rel_table, triples):
    wall = _phase_a(triples[:, 0], triples[:, 1], triples[:, 2])
    return _phase_b(memory, rel_table, wall)


# speedup vs baseline: 1.0163x; 1.0047x over previous
import functools

import jax
import jax.numpy as jnp
from jax import lax
from jax.experimental import pallas as pl
from jax.experimental.pallas import tpu as pltpu
from jax.experimental.pallas import tpu_sc as plsc

NE = 100000
NR = 1000
D = 64
NT = 1600000

NW = 32
CHUNK = NT // NW
HWIN = 2000
NHW = CHUNK // HWIN

EMPTY = 0x7FFFFFFF
SIGN = -2147483648

MA_BIG = 10
MA_BIG_N, MA_SMALL_N = 6256, 6240
MA_BIG_TOTAL = MA_BIG * MA_BIG_N

N_BIG = 10
BIG_N, BIG_W, BIG_WINS = 3136, 224, 14
SMALL_N, SMALL_W, SMALL_WINS = 3120, 240, 13
BIG_TOTAL = N_BIG * BIG_N
SPP = 248

_MESH = plsc.VectorSubcoreMesh(core_axis_name="c", subcore_axis_name="s")
_PARAMS = pltpu.CompilerParams(needs_layout_passes=False,
                               use_tc_tiling_on_sc=False)


@functools.partial(
    pl.kernel,
    out_type=jax.ShapeDtypeStruct((NW * NE,), jnp.int32),
    mesh=_MESH,
    compiler_params=_PARAMS,
    scratch_types=[
        pltpu.VMEM((NE,), jnp.int32),
        pltpu.VMEM((3 * HWIN,), jnp.int32),
        pltpu.VMEM((3 * HWIN,), jnp.int32),
        pltpu.SemaphoreType.DMA,
        pltpu.SemaphoreType.DMA,
    ],
)
def _phase_a(h_hbm, r_hbm, t_hbm, wall_hbm, table, bufA, bufB, semA, semB):
    cid = lax.axis_index("c")
    sid = lax.axis_index("s")
    wid = sid * 2 + cid
    lanes = lax.iota(jnp.int32, 16)

    def init(i, carry):
        table[pl.ds(i * 16, 16)] = jnp.full((16,), EMPTY, jnp.int32)
        return carry

    lax.fori_loop(0, NE // 16, init, 0)

    base = wid * CHUNK
    widtag = (31 - wid) << 27

    def start_win(w, buf, sem):
        s = base + w * HWIN
        for c, col in enumerate((h_hbm, r_hbm, t_hbm)):
            pltpu.async_copy(col.at[pl.ds(s, HWIN)],
                             buf.at[pl.ds(c * HWIN, HWIN)], sem)

    def wait_win(buf, sem):
        for c in range(3):
            pltpu.make_async_copy(h_hbm.at[pl.ds(base, HWIN)],
                                  buf.at[pl.ds(c * HWIN, HWIN)], sem).wait()

    def scan_buf(w, buf):
        def vstep(i, c2):
            sl = pl.ds(i * 16, 16)
            h = buf[sl]
            rl = buf[pl.ds(HWIN + i * 16, 16)]
            tl = buf[pl.ds(2 * HWIN + i * 16, 16)]
            key = (h << 4) | lanes
            pay = (rl << 17) | tl
            ks, vs = plsc.sort_key_val(key, pay)
            hs = ks >> 4
            nxt = hs.at[jnp.minimum(lanes + 1, 15)].get(
                mode="promise_in_bounds")
            mask = (hs != nxt) | (lanes == 15)
            val = (widtag | vs) ^ SIGN
            plsc.store_scatter(table, [hs], val, mask=mask)
            return c2

        lax.fori_loop(0, HWIN // 16, vstep, 0, unroll=16)

    start_win(0, bufA, semA)

    def win2(w, carry):
        @pl.when(w % 2 == 0)
        def _even():
            wait_win(bufA, semA)

            @pl.when(w + 1 < NHW)
            def _():
                start_win(w + 1, bufB, semB)

            scan_buf(w, bufA)

        @pl.when(w % 2 == 1)
        def _odd():
            wait_win(bufB, semB)

            @pl.when(w + 1 < NHW)
            def _():
                start_win(w + 1, bufA, semA)

            scan_buf(w, bufB)

        return carry

    lax.fori_loop(0, NHW, win2, 0)
    pltpu.sync_copy(table, wall_hbm.at[pl.ds(wid * NE, NE)])


@functools.partial(
    pl.kernel,
    out_type=jax.ShapeDtypeStruct((NE, D), jnp.float32),
    mesh=_MESH,
    compiler_params=_PARAMS,
    scratch_types=[
        pltpu.VMEM((BIG_N,), jnp.int32),
        pltpu.VMEM((BIG_N,), jnp.int32),
        pltpu.VMEM((BIG_N,), jnp.int32),
        pltpu.VMEM((BIG_W,), jnp.int32),
        pltpu.VMEM((BIG_W,), jnp.int32),
        pltpu.VMEM((SMALL_W,), jnp.int32),
        pltpu.VMEM((SMALL_W,), jnp.int32),
        pltpu.VMEM((BIG_W,), jnp.int32),
        pltpu.VMEM((BIG_W,), jnp.int32),
        pltpu.VMEM((SMALL_W,), jnp.int32),
        pltpu.VMEM((SMALL_W,), jnp.int32),
        pltpu.VMEM((BIG_W,), jnp.int32),
        pltpu.VMEM((BIG_W,), jnp.int32),
        pltpu.VMEM((SMALL_W,), jnp.int32),
        pltpu.VMEM((SMALL_W,), jnp.int32),
        pltpu.VMEM((SMALL_W, D), jnp.float32),
        pltpu.VMEM((SMALL_W, D), jnp.float32),
        pltpu.VMEM((SMALL_W, D), jnp.float32),
        pltpu.VMEM((SMALL_W, D), jnp.float32),
        pltpu.VMEM_SHARED((16 * 2 * SPP, D), jnp.float32),
        pltpu.SemaphoreType.DMA,
        pltpu.SemaphoreType.DMA,
        pltpu.SemaphoreType.DMA,
        pltpu.SemaphoreType.DMA,
        pltpu.SemaphoreType.DMA,
        pltpu.SemaphoreType.DMA,
        pltpu.SemaphoreType.DMA,
        pltpu.SemaphoreType.DMA,
    ],
)
def _phase_b(mem_hbm, rel_hbm, wall_hbm, out_hbm,
             wfull, stg0, stg1, tgtA0, tgtA1, tgtB0, tgtB1,
             rwA0, rwA1, rwB0, rwB1, twA0, twA1, twB0, twB1,
             rel0, rel1, tb0, tb1, spmem,
             semr0, semr1, semt0, semt1, semm0, semm1, semo0, semo1):
    cid = lax.axis_index("c")
    sid = lax.axis_index("s")
    wid = sid * 2 + cid
    lanes = lax.iota(jnp.int32, 16)

    rels = (rel0, rel1)
    tbs = (tb0, tb1)
    semrs = (semr0, semr1)
    semts = (semt0, semt1)
    semms = (semm0, semm1)
    semos = (semo0, semo1)

    def run(nwin, wsize, e0_fn, tgts, rws, tws):
        nv = wsize // 16
        n_range = nwin * wsize
        e0r = e0_fn(0)
        stgs = (stg0, stg1)

        cw = pltpu.async_copy(wall_hbm.at[pl.ds(e0r, n_range)],
                              wfull.at[pl.ds(0, n_range)], semt0)
        ck = pltpu.async_copy(wall_hbm.at[pl.ds(NE + e0r, n_range)],
                              stg0.at[pl.ds(0, n_range)], semr0)
        cw.wait()
        for k in range(1, NW):
            p = (k - 1) % 2
            if k + 1 < NW:
                pltpu.async_copy(
                    wall_hbm.at[pl.ds((k + 1) * NE + e0r, n_range)],
                    stgs[1 - p].at[pl.ds(0, n_range)],
                    semrs[1 - p])
            if k == 1:
                ck.wait()
            else:
                pltpu.make_async_copy(
                    wall_hbm.at[pl.ds(k * NE + e0r, n_range)],
                    stgs[p].at[pl.ds(0, n_range)], semrs[p]).wait()
            stg = stgs[p]

            def mstep(j, carry):
                col = pl.ds(j * 16, 16)
                wfull[col] = jnp.minimum(wfull[col], stg[col])
                return carry

            lax.fori_loop(0, n_range // 16, mstep, 0, unroll=4)

        out_h = {}

        def stage1(w):
            p = w % 2
            e0 = e0_fn(w)
            spb = sid * (2 * SPP) + p * SPP
            rw, tw, tgt = rws[p], tws[p], tgts[p]
            if w - 2 in out_h:
                out_h[w - 2].wait()

            def jstep(j, carry):
                col = pl.ds(j * 16, 16)
                acc = wfull[pl.ds(w * wsize + j * 16, 16)]
                nowin = acc == EMPTY
                u = acc ^ SIGN
                r = lax.shift_right_logical(u, 17) & 0x3FF
                t = u & 0x1FFFF
                rw[col] = jnp.where(nowin, 0, r)
                tw[col] = jnp.where(nowin, 0, t)
                eloc = j * 16 + lanes
                tgt[col] = spb + jnp.where(nowin, 240 + (lanes & 7), eloc)
                return carry

            lax.fori_loop(0, nv, jstep, 0, unroll=2)

            cr = pltpu.async_copy(rel_hbm.at[rw],
                                  rels[p].at[pl.ds(0, wsize)], semrs[p])
            ct = pltpu.async_copy(mem_hbm.at[tw],
                                  tbs[p].at[pl.ds(0, wsize)], semts[p])
            cm = pltpu.async_copy(mem_hbm.at[pl.ds(e0, wsize)],
                                  spmem.at[pl.ds(spb, wsize)], semms[p])
            return (cr, ct, cm)

        def stage2(w, handles):
            p = w % 2
            e0 = e0_fn(w)
            spb = sid * (2 * SPP) + p * SPP
            rel, tb, tgt = rels[p], tbs[p], tgts[p]
            for c in handles:
                c.wait()

            def dstep(j, carry):
                for c in range(D // 16):
                    col = pl.ds(c * 16, 16)
                    rel[j, col] = 0.1 * (rel[j, col] - tb[j, col])
                return carry

            lax.fori_loop(0, wsize, dstep, 0, unroll=2)

            pltpu.sync_copy(rel.at[pl.ds(0, wsize)], spmem.at[tgt],
                            add=True)
            out_h[w] = pltpu.async_copy(spmem.at[pl.ds(spb, wsize)],
                                        out_hbm.at[pl.ds(e0, wsize)],
                                        semos[p])

        handles = stage1(0)
        for w in range(1, nwin):
            nh = stage1(w)
            stage2(w - 1, handles)
            handles = nh
        stage2(nwin - 1, handles)
        out_h[nwin - 2].wait()
        out_h[nwin - 1].wait()

    @pl.when(wid < N_BIG)
    def _big():
        run(BIG_WINS, BIG_W, lambda w: wid * BIG_N + w * BIG_W,
            (tgtA0, tgtA1), (rwA0, rwA1), (twA0, twA1))

    @pl.when(wid >= N_BIG)
    def _small():
        run(SMALL_WINS, SMALL_W,
            lambda w: BIG_TOTAL + (wid - N_BIG) * SMALL_N + w * SMALL_W,
            (tgtB0, tgtB1), (rwB0, rwB1), (twB0, twB1))


def kernel(memory, rel_table, triples):
    wall = _phase_a(triples[:, 0], triples[:, 1], triples[:, 2])
    return _phase_b(memory, rel_table, wall)


# speedup vs baseline: 1.0170x; 1.0007x over previous
import functools

import jax
import jax.numpy as jnp
from jax import lax
from jax.experimental import pallas as pl
from jax.experimental.pallas import tpu as pltpu
from jax.experimental.pallas import tpu_sc as plsc

NE = 100000
NR = 1000
D = 64
NT = 1600000

NW = 32
CHUNK = NT // NW
HWIN = 2000
NHW = CHUNK // HWIN

EMPTY = 0x7FFFFFFF
SIGN = -2147483648

N_BIG = 10
BIG_N, BIG_W, BIG_WINS = 3136, 224, 14
SMALL_N, SMALL_W, SMALL_WINS = 3120, 240, 13
BIG_TOTAL = N_BIG * BIG_N
SPP = 248

_MESH = plsc.VectorSubcoreMesh(core_axis_name="c", subcore_axis_name="s")
_PARAMS = pltpu.CompilerParams(needs_layout_passes=False,
                               use_tc_tiling_on_sc=False)


@functools.partial(
    pl.kernel,
    out_type=jax.ShapeDtypeStruct((NW * NE,), jnp.int32),
    mesh=_MESH,
    compiler_params=_PARAMS,
    scratch_types=[
        pltpu.VMEM((NE,), jnp.int32),
        pltpu.VMEM((3 * HWIN,), jnp.int32),
        pltpu.VMEM((3 * HWIN,), jnp.int32),
        pltpu.SemaphoreType.DMA,
        pltpu.SemaphoreType.DMA,
    ],
)
def _phase_a(h_hbm, r_hbm, t_hbm, wall_hbm, table, bufA, bufB, semA, semB):
    cid = lax.axis_index("c")
    sid = lax.axis_index("s")
    wid = sid * 2 + cid
    lanes = lax.iota(jnp.int32, 16)

    def init(i, carry):
        table[pl.ds(i * 16, 16)] = jnp.full((16,), EMPTY, jnp.int32)
        return carry

    lax.fori_loop(0, NE // 16, init, 0)

    base = wid * CHUNK
    widtag = (31 - wid) << 27

    def start_win(w, buf, sem):
        s = base + w * HWIN
        for c, col in enumerate((h_hbm, r_hbm, t_hbm)):
            pltpu.async_copy(col.at[pl.ds(s, HWIN)],
                             buf.at[pl.ds(c * HWIN, HWIN)], sem)

    def wait_win(buf, sem):
        for c in range(3):
            pltpu.make_async_copy(h_hbm.at[pl.ds(base, HWIN)],
                                  buf.at[pl.ds(c * HWIN, HWIN)], sem).wait()

    def scan_buf(w, buf):
        def vstep(i, c2):
            sl = pl.ds(i * 16, 16)
            h = buf[sl]
            rl = buf[pl.ds(HWIN + i * 16, 16)]
            tl = buf[pl.ds(2 * HWIN + i * 16, 16)]
            key = (h << 4) | lanes
            pay = (rl << 17) | tl
            ks, vs = plsc.sort_key_val(key, pay)
            hs = ks >> 4
            nxt = hs.at[jnp.minimum(lanes + 1, 15)].get(
                mode="promise_in_bounds")
            mask = (hs != nxt) | (lanes == 15)
            val = (widtag | vs) ^ SIGN
            plsc.store_scatter(table, [hs], val, mask=mask)
            return c2

        lax.fori_loop(0, HWIN // 16, vstep, 0, unroll=16)

    start_win(0, bufA, semA)

    def win2(w, carry):
        @pl.when(w % 2 == 0)
        def _even():
            wait_win(bufA, semA)

            @pl.when(w + 1 < NHW)
            def _():
                start_win(w + 1, bufB, semB)

            scan_buf(w, bufA)

        @pl.when(w % 2 == 1)
        def _odd():
            wait_win(bufB, semB)

            @pl.when(w + 1 < NHW)
            def _():
                start_win(w + 1, bufA, semA)

            scan_buf(w, bufB)

        return carry

    lax.fori_loop(0, NHW, win2, 0)
    pltpu.sync_copy(table, wall_hbm.at[pl.ds(wid * NE, NE)])


@functools.partial(
    pl.kernel,
    out_type=jax.ShapeDtypeStruct((NE, D), jnp.float32),
    mesh=_MESH,
    compiler_params=_PARAMS,
    scratch_types=[
        pltpu.VMEM((BIG_N,), jnp.int32),
        pltpu.VMEM((BIG_N,), jnp.int32),
        pltpu.VMEM((BIG_N,), jnp.int32),
        pltpu.VMEM((BIG_W,), jnp.int32),
        pltpu.VMEM((BIG_W,), jnp.int32),
        pltpu.VMEM((SMALL_W,), jnp.int32),
        pltpu.VMEM((SMALL_W,), jnp.int32),
        pltpu.VMEM((BIG_W,), jnp.int32),
        pltpu.VMEM((BIG_W,), jnp.int32),
        pltpu.VMEM((SMALL_W,), jnp.int32),
        pltpu.VMEM((SMALL_W,), jnp.int32),
        pltpu.VMEM((BIG_W,), jnp.int32),
        pltpu.VMEM((BIG_W,), jnp.int32),
        pltpu.VMEM((SMALL_W,), jnp.int32),
        pltpu.VMEM((SMALL_W,), jnp.int32),
        pltpu.VMEM((SMALL_W, D), jnp.float32),
        pltpu.VMEM((SMALL_W, D), jnp.float32),
        pltpu.VMEM((SMALL_W, D), jnp.float32),
        pltpu.VMEM((SMALL_W, D), jnp.float32),
        pltpu.VMEM_SHARED((16 * 2 * SPP, D), jnp.float32),
        pltpu.SemaphoreType.DMA,
        pltpu.SemaphoreType.DMA,
        pltpu.SemaphoreType.DMA,
        pltpu.SemaphoreType.DMA,
        pltpu.SemaphoreType.DMA,
        pltpu.SemaphoreType.DMA,
        pltpu.SemaphoreType.DMA,
        pltpu.SemaphoreType.DMA,
    ],
)
def _phase_b(mem_hbm, rel_hbm, wall_hbm, out_hbm,
             wfull, stg0, stg1, tgtA0, tgtA1, tgtB0, tgtB1,
             rwA0, rwA1, rwB0, rwB1, twA0, twA1, twB0, twB1,
             rel0, rel1, tb0, tb1, spmem,
             semr0, semr1, semt0, semt1, semm0, semm1, semo0, semo1):
    cid = lax.axis_index("c")
    sid = lax.axis_index("s")
    wid = sid * 2 + cid
    lanes = lax.iota(jnp.int32, 16)

    rels = (rel0, rel1)
    tbs = (tb0, tb1)
    semrs = (semr0, semr1)
    semts = (semt0, semt1)
    semms = (semm0, semm1)
    semos = (semo0, semo1)

    def run(nwin, wsize, e0_fn, tgts, rws, tws):
        nv = wsize // 16
        n_range = nwin * wsize
        e0r = e0_fn(0)
        stgs = (stg0, stg1)

        cw = pltpu.async_copy(wall_hbm.at[pl.ds(e0r, n_range)],
                              wfull.at[pl.ds(0, n_range)], semt0)
        ck = pltpu.async_copy(wall_hbm.at[pl.ds(NE + e0r, n_range)],
                              stg0.at[pl.ds(0, n_range)], semr0)
        cw.wait()
        for k in range(1, NW):
            p = (k - 1) % 2
            if k + 1 < NW:
                pltpu.async_copy(
                    wall_hbm.at[pl.ds((k + 1) * NE + e0r, n_range)],
                    stgs[1 - p].at[pl.ds(0, n_range)],
                    semrs[1 - p])
            if k == 1:
                ck.wait()
            else:
                pltpu.make_async_copy(
                    wall_hbm.at[pl.ds(k * NE + e0r, n_range)],
                    stgs[p].at[pl.ds(0, n_range)], semrs[p]).wait()
            stg = stgs[p]

            def mstep(j, carry):
                col = pl.ds(j * 16, 16)
                wfull[col] = jnp.minimum(wfull[col], stg[col])
                return carry

            lax.fori_loop(0, n_range // 16, mstep, 0, unroll=4)

        out_h = {}

        def stage1(w):
            p = w % 2
            e0 = e0_fn(w)
            spb = sid * (2 * SPP) + p * SPP
            rw, tw, tgt = rws[p], tws[p], tgts[p]
            if w - 2 in out_h:
                out_h[w - 2].wait()

            def jstep(j, carry):
                col = pl.ds(j * 16, 16)
                acc = wfull[pl.ds(w * wsize + j * 16, 16)]
                nowin = acc == EMPTY
                u = acc ^ SIGN
                r = lax.shift_right_logical(u, 17) & 0x3FF
                t = u & 0x1FFFF
                rw[col] = jnp.where(nowin, 0, r)
                tw[col] = jnp.where(nowin, 0, t)
                eloc = j * 16 + lanes
                tgt[col] = spb + jnp.where(nowin, 240 + (lanes & 7), eloc)
                return carry

            lax.fori_loop(0, nv, jstep, 0, unroll=2)

            cr = pltpu.async_copy(rel_hbm.at[rw],
                                  rels[p].at[pl.ds(0, wsize)], semrs[p])
            ct = pltpu.async_copy(mem_hbm.at[tw],
                                  tbs[p].at[pl.ds(0, wsize)], semts[p])
            cm = pltpu.async_copy(mem_hbm.at[pl.ds(e0, wsize)],
                                  spmem.at[pl.ds(spb, wsize)], semms[p])
            return (cr, ct, cm)

        def stage2(w, handles):
            p = w % 2
            e0 = e0_fn(w)
            spb = sid * (2 * SPP) + p * SPP
            rel, tb, tgt = rels[p], tbs[p], tgts[p]
            for c in handles:
                c.wait()

            def dstep(j, carry):
                for c in range(D // 16):
                    col = pl.ds(c * 16, 16)
                    rel[j, col] = 0.1 * (rel[j, col] - tb[j, col])
                return carry

            lax.fori_loop(0, wsize, dstep, 0, unroll=2)

            pltpu.sync_copy(rel.at[pl.ds(0, wsize)], spmem.at[tgt],
                            add=True)
            out_h[w] = pltpu.async_copy(spmem.at[pl.ds(spb, wsize)],
                                        out_hbm.at[pl.ds(e0, wsize)],
                                        semos[p])

        handles = stage1(0)
        for w in range(1, nwin):
            nh = stage1(w)
            stage2(w - 1, handles)
            handles = nh
        stage2(nwin - 1, handles)
        out_h[nwin - 2].wait()
        out_h[nwin - 1].wait()

    @pl.when(wid < N_BIG)
    def _big():
        run(BIG_WINS, BIG_W, lambda w: wid * BIG_N + w * BIG_W,
            (tgtA0, tgtA1), (rwA0, rwA1), (twA0, twA1))

    @pl.when(wid >= N_BIG)
    def _small():
        run(SMALL_WINS, SMALL_W,
            lambda w: BIG_TOTAL + (wid - N_BIG) * SMALL_N + w * SMALL_W,
            (tgtB0, tgtB1), (rwB0, rwB1), (twB0, twB1))


def kernel(memory, rel_table, triples):
    wall = _phase_a(triples[:, 0], triples[:, 1], triples[:, 2])
    return _phase_b(memory, rel_table, wall)
